# baseline (speedup 1.0000x reference)
import functools

import jax
import jax.numpy as jnp
from jax import lax
from jax.experimental import pallas as pl
from jax.experimental.pallas import tpu as pltpu
from jax.experimental.pallas import tpu_sc as plsc

DIM = 17
TBL = DIM ** 4
CHUNK = 2048
SUB = 128
NSUB = CHUNK // SUB
NGRP = SUB // 16
PX_PER_TILE = 65536
NCHUNK = PX_PER_TILE // CHUNK
OFF = (0, DIM ** 2, DIM ** 3, DIM ** 3 + DIM ** 2)


def _build_table(LUT):
    lutT = jnp.transpose(LUT[0], (1, 2, 3, 4, 0))
    Xp = jnp.pad(lutT, ((0, 0), (0, 0), (0, 1), (0, 1), (0, 0)))
    rows = jnp.stack(
        [Xp[:, :, (n & 1):(n & 1) + DIM, (n >> 1):(n >> 1) + DIM]
         for n in range(4)],
        axis=4)
    return rows.reshape(TBL, 16)


def kernel(x, LUT):
    B, C, H, W = x.shape
    xr = x.reshape(B, C, H * W)
    table = _build_table(LUT)
    info = plsc.get_sparse_core_info()
    NC = info.num_cores

    mesh = plsc.VectorSubcoreMesh(core_axis_name="c", subcore_axis_name="s")

    @functools.partial(
        pl.kernel,
        mesh=mesh,
        out_type=jax.ShapeDtypeStruct((B, C, H * W), jnp.float32),
        scratch_types=[
            pltpu.VMEM((4, CHUNK), jnp.float32),
            pltpu.VMEM((4, CHUNK), jnp.float32),
            pltpu.VMEM((2, CHUNK), jnp.float32),
            pltpu.VMEM((CHUNK * 16,), jnp.float32),
            pltpu.VMEM((NSUB, 4, SUB), jnp.int32),
            pltpu.VMEM((2, SUB * 4, 16), jnp.float32),
            pltpu.VMEM((CHUNK * 4,), jnp.float32),
            pltpu.SemaphoreType.DMA,
            pltpu.SemaphoreType.DMA,
            pltpu.SemaphoreType.DMA,
        ],
        compiler_params=pltpu.CompilerParams(
            needs_layout_passes=False, use_tc_tiling_on_sc=False),
    )
    def sc_kernel(x_hbm, tbl_hbm, out_hbm, xbuf, outbuf, dbuf, wbuf, idxbuf,
                  rowbuf, aosbuf, semx, semg0, semg1):
        wid = lax.axis_index("s") * NC + lax.axis_index("c")
        b = wid // 4
        base = (wid % 4) * PX_PER_TILE
        iota = lax.iota(jnp.int32, 16)
        rot4 = (iota + 4) & 15
        rot8 = (iota + 8) & 15
        m4 = iota < 4
        cidx = [jnp.full((16,), i, jnp.int32) for i in range(16)]

        def fire(j, p, sem):
            for m in range(4):
                pltpu.async_copy(tbl_hbm.at[idxbuf.at[j, m]],
                                 rowbuf.at[p, pl.ds(m * SUB, SUB)], sem)

        def drain(p, sem):
            pltpu.make_async_copy(tbl_hbm.at[idxbuf.at[0, 0]], rowbuf.at[p],
                                  sem).wait()

        def compute_sub(j, p):
            def grp_body(k, _):
                qv = j * SUB + k * 16
                dv0 = dbuf[0, pl.ds(qv, 16)]
                dv1 = dbuf[1, pl.ds(qv, 16)]
                ao = qv * 4 + iota
                for pix in range(16):
                    r = k * 16 + pix
                    v0 = rowbuf[p, r, :]
                    v1 = rowbuf[p, SUB + r, :]
                    v2 = rowbuf[p, 2 * SUB + r, :]
                    v3 = rowbuf[p, 3 * SUB + r, :]
                    d0 = jnp.take(dv0, cidx[pix], mode="wrap")
                    d1 = jnp.take(dv1, cidx[pix], mode="wrap")
                    wrow = wbuf[pl.ds((qv + pix) * 16, 16)]
                    a0 = v0 + (v1 - v0) * d1
                    a1 = v2 + (v3 - v2) * d1
                    s = (a0 + (a1 - a0) * d0) * wrow
                    s = s + jnp.take(s, rot8, mode="wrap")
                    s = s + jnp.take(s, rot4, mode="wrap")
                    plsc.store_scatter(aosbuf, [ao + pix * 4], s, mask=m4)
                return _
            lax.fori_loop(0, NGRP, grp_body, None)

        def chunk_body(g, _):
            start = base + g * CHUNK
            xd = [pltpu.async_copy(x_hbm.at[b, c, pl.ds(start, CHUNK)],
                                   xbuf.at[c], semx) for c in range(4)]
            for dsc in xd:
                dsc.wait()

            def idx_body(j, _):
                def grp_body(k, _):
                    q = j * SUB + k * 16
                    idx = None
                    dd = []
                    for c in range(4):
                        xv = xbuf[c, pl.ds(q, 16)]
                        xv = jnp.minimum(jnp.maximum(xv, 0.0), 1.0)
                        posv = xv * jnp.float32(DIM - 1)
                        fi = jnp.minimum(posv.astype(jnp.int32), DIM - 2)
                        dd.append(posv - fi.astype(jnp.float32))
                        idx = fi if idx is None else idx * DIM + fi
                    dbuf[0, pl.ds(q, 16)] = dd[0]
                    dbuf[1, pl.ds(q, 16)] = dd[1]
                    for m in range(4):
                        idxbuf[j, m, pl.ds(k * 16, 16)] = idx + OFF[m]
                    e2, d2v, e3, d3v = 1.0 - dd[2], dd[2], 1.0 - dd[3], dd[3]
                    q00 = e2 * e3
                    q10 = d2v * e3
                    q01 = e2 * d3v
                    q11 = d2v * d3v
                    qs = (q00, q10, q01, q11)
                    wo = q * 16 + iota * 16
                    for ln in range(16):
                        wv = qs[((ln >> 2) & 1) + 2 * ((ln >> 3) & 1)]
                        plsc.store_scatter(wbuf, [wo + ln], wv)
                    return _
                return lax.fori_loop(0, NGRP, grp_body, _)
            lax.fori_loop(0, NSUB, idx_body, None)

            fire(0, 0, semg0)

            def pair_body(t, _):
                j0 = 2 * t
                fire(j0 + 1, 1, semg1)
                drain(0, semg0)
                compute_sub(j0, 0)
                fire(j0 + 2, 0, semg0)
                drain(1, semg1)
                compute_sub(j0 + 1, 1)
                return _
            lax.fori_loop(0, NSUB // 2 - 1, pair_body, None)
            fire(NSUB - 1, 1, semg1)
            drain(0, semg0)
            compute_sub(NSUB - 2, 0)
            drain(1, semg1)
            compute_sub(NSUB - 1, 1)

            def tr_body(k, _):
                addr = (k * 16 + iota) * 4
                for c in range(4):
                    outbuf[c, pl.ds(k * 16, 16)] = plsc.load_gather(
                        aosbuf, [addr + c])
                return _
            lax.fori_loop(0, CHUNK // 16, tr_body, None)

            for c in range(4):
                pltpu.sync_copy(outbuf.at[c],
                                out_hbm.at[b, c, pl.ds(start, CHUNK)])
            return _
        lax.fori_loop(0, NCHUNK, chunk_body, None)

    out = sc_kernel(xr, table)
    return out.reshape(B, C, H, W)


# speedup vs baseline: 1.9306x; 1.9306x over previous
import functools

import jax
import jax.numpy as jnp
from jax import lax
from jax.experimental import pallas as pl
from jax.experimental.pallas import tpu as pltpu
from jax.experimental.pallas import tpu_sc as plsc

DIM = 17
TBL = DIM ** 4
CHUNK = 2048
SUB = 128
NSUB = CHUNK // SUB
NGRP = SUB // 16
PX_PER_TILE = 65536
NCHUNK = PX_PER_TILE // CHUNK
OFF = (0, DIM ** 2, DIM ** 3, DIM ** 3 + DIM ** 2)


def _build_table(LUT):
    lutT = jnp.transpose(LUT[0], (1, 2, 3, 4, 0))
    Xp = jnp.pad(lutT, ((0, 0), (0, 0), (0, 1), (0, 1), (0, 0)))
    rows = jnp.stack(
        [Xp[:, :, (n & 1):(n & 1) + DIM, (n >> 1):(n >> 1) + DIM]
         for n in range(4)],
        axis=4)
    return rows.reshape(TBL, 16)


def kernel(x, LUT):
    B, C, H, W = x.shape
    xr = x.reshape(B, C, H * W)
    table = _build_table(LUT)
    info = plsc.get_sparse_core_info()
    NC = info.num_cores

    mesh = plsc.VectorSubcoreMesh(core_axis_name="c", subcore_axis_name="s")

    @functools.partial(
        pl.kernel,
        mesh=mesh,
        out_type=jax.ShapeDtypeStruct((B, C, H * W), jnp.float32),
        scratch_types=[
            pltpu.VMEM((4, CHUNK), jnp.float32),
            pltpu.VMEM((4, CHUNK), jnp.float32),
            pltpu.VMEM((2, CHUNK), jnp.float32),
            pltpu.VMEM((CHUNK * 16,), jnp.float32),
            pltpu.VMEM((NSUB, 4, SUB), jnp.int32),
            pltpu.VMEM((2, SUB * 4, 16), jnp.float32),
            pltpu.VMEM((CHUNK * 4,), jnp.float32),
            pltpu.SemaphoreType.DMA,
            pltpu.SemaphoreType.DMA,
            pltpu.SemaphoreType.DMA,
        ],
        compiler_params=pltpu.CompilerParams(
            needs_layout_passes=False, use_tc_tiling_on_sc=False),
    )
    def sc_kernel(x_hbm, tbl_hbm, out_hbm, xbuf, outbuf, dbuf, wbuf, idxbuf,
                  rowbuf, aosbuf, semx, semg0, semg1):
        wid = lax.axis_index("s") * NC + lax.axis_index("c")
        b = wid // 4
        base = (wid % 4) * PX_PER_TILE
        iota = lax.iota(jnp.int32, 16)
        rot4 = (iota + 4) & 15
        rot8 = (iota + 8) & 15
        m4 = iota < 4
        iota16 = iota * 16

        def fire(j, p, sem):
            for m in range(4):
                pltpu.async_copy(tbl_hbm.at[idxbuf.at[j, m]],
                                 rowbuf.at[p, pl.ds(m * SUB, SUB)], sem)

        def drain(p, sem):
            pltpu.make_async_copy(tbl_hbm.at[idxbuf.at[0, 0]], rowbuf.at[p],
                                  sem).wait()

        def compute_sub(j, p):
            def grp_body(k, _):
                qv = j * SUB + k * 16
                dv0 = dbuf[0, pl.ds(qv, 16)]
                dv1 = dbuf[1, pl.ds(qv, 16)]
                ao = qv * 4 + iota
                w0 = qv * 16

                @plsc.parallel_loop(0, 16, unroll=4)
                def px_body(pix):
                    r = k * 16 + pix
                    v0 = rowbuf[p, r, :]
                    v1 = rowbuf[p, SUB + r, :]
                    v2 = rowbuf[p, 2 * SUB + r, :]
                    v3 = rowbuf[p, 3 * SUB + r, :]
                    bidx = jnp.full((16,), pix, jnp.int32)
                    d0 = jnp.take(dv0, bidx, mode="wrap")
                    d1 = jnp.take(dv1, bidx, mode="wrap")
                    wrow = wbuf[pl.ds(w0 + pix * 16, 16)]
                    a0 = v0 + (v1 - v0) * d1
                    a1 = v2 + (v3 - v2) * d1
                    s = (a0 + (a1 - a0) * d0) * wrow
                    s = s + jnp.take(s, rot8, mode="wrap")
                    s = s + jnp.take(s, rot4, mode="wrap")
                    plsc.store_scatter(aosbuf, [ao + pix * 4], s, mask=m4)
                return _
            lax.fori_loop(0, NGRP, grp_body, None)

        def chunk_body(g, _):
            start = base + g * CHUNK
            xd = [pltpu.async_copy(x_hbm.at[b, c, pl.ds(start, CHUNK)],
                                   xbuf.at[c], semx) for c in range(4)]
            for dsc in xd:
                dsc.wait()

            def idx_body(j, _):
                @plsc.parallel_loop(0, NGRP, unroll=2)
                def grp_body(k):
                    q = j * SUB + k * 16
                    idx = None
                    dd = []
                    for c in range(4):
                        xv = xbuf[c, pl.ds(q, 16)]
                        xv = jnp.minimum(jnp.maximum(xv, 0.0), 1.0)
                        posv = xv * jnp.float32(DIM - 1)
                        fi = jnp.minimum(posv.astype(jnp.int32), DIM - 2)
                        dd.append(posv - fi.astype(jnp.float32))
                        idx = fi if idx is None else idx * DIM + fi
                    dbuf[0, pl.ds(q, 16)] = dd[0]
                    dbuf[1, pl.ds(q, 16)] = dd[1]
                    for m in range(4):
                        idxbuf[j, m, pl.ds(k * 16, 16)] = idx + OFF[m]
                    e2, d2v, e3, d3v = 1.0 - dd[2], dd[2], 1.0 - dd[3], dd[3]
                    q00 = e2 * e3
                    q10 = d2v * e3
                    q01 = e2 * d3v
                    q11 = d2v * d3v
                    qs = (q00, q10, q01, q11)
                    wo = q * 16 + iota16
                    for ln in range(16):
                        wv = qs[((ln >> 2) & 1) + 2 * ((ln >> 3) & 1)]
                        plsc.store_scatter(wbuf, [wo + ln], wv)
                return _
            lax.fori_loop(0, NSUB, idx_body, None)

            fire(0, 0, semg0)

            def pair_body(t, _):
                j0 = 2 * t
                fire(j0 + 1, 1, semg1)
                drain(0, semg0)
                compute_sub(j0, 0)
                fire(j0 + 2, 0, semg0)
                drain(1, semg1)
                compute_sub(j0 + 1, 1)
                return _
            lax.fori_loop(0, NSUB // 2 - 1, pair_body, None)
            fire(NSUB - 1, 1, semg1)
            drain(0, semg0)
            compute_sub(NSUB - 2, 0)
            drain(1, semg1)
            compute_sub(NSUB - 1, 1)

            @plsc.parallel_loop(0, CHUNK // 16, unroll=2)
            def tr_body(k):
                addr = (k * 16 + iota) * 4
                for c in range(4):
                    outbuf[c, pl.ds(k * 16, 16)] = plsc.load_gather(
                        aosbuf, [addr + c])

            for c in range(4):
                pltpu.sync_copy(outbuf.at[c],
                                out_hbm.at[b, c, pl.ds(start, CHUNK)])
            return _
        lax.fori_loop(0, NCHUNK, chunk_body, None)

    out = sc_kernel(xr, table)
    return out.reshape(B, C, H, W)
